# baseline (speedup 1.0000x reference)
import functools

import jax
import jax.numpy as jnp
from jax import lax
from jax.experimental import pallas as pl
from jax.experimental.pallas import tpu as pltpu
from jax.experimental.pallas import tpu_sc as plsc

N = 10000
NS = 5000
H = 128
HS = 64
E = 320000
ES = 50000
L = 3

NC = 2
NSUB = 16
NW = NC * NSUB
CH = 128
KB = 1
BLK = 16

NCH_E = 80
NCH_ES = 16
E_PAD = NW * NCH_E * CH
ES_PAD = NW * NCH_ES * CH

N_WB = 10112
NS_WB = 5120
N_ACC = 10240


def _make_segsum(nch, rpad, n_wb):
  mesh = plsc.VectorSubcoreMesh(core_axis_name="c", subcore_axis_name="s")
  rz = rpad // NW
  rwb = n_wb // NSUB
  nblk = nch // BLK

  @functools.partial(
      pl.kernel,
      out_type=jax.ShapeDtypeStruct((NC, n_wb, H), jnp.float32),
      mesh=mesh,
      scratch_types=(
          [pltpu.VMEM((BLK, CH), jnp.int32),
           pltpu.VMEM((BLK, CH), jnp.int32)]
          + [pltpu.VMEM((CH, H), jnp.float32) for _ in range(KB)]
          + [pltpu.VMEM((16, H), jnp.float32)]
          + [pltpu.VMEM_SHARED((rpad, H), jnp.float32)]
          + [pltpu.SemaphoreType.DMA for _ in range(2 * KB)]
      ),
  )
  def seg(table, gidx, sidx, out, *bufs):
    gbB = bufs[0]
    sbB = bufs[1]
    rows = bufs[2:KB + 2]
    zb = bufs[KB + 2]
    acc = bufs[KB + 3]
    gsem = bufs[KB + 4:2 * KB + 4]
    ssem = bufs[2 * KB + 4:]
    c = lax.axis_index("c")
    s = lax.axis_index("s")
    w = c * NSUB + s
    zero16 = jnp.zeros((16,), jnp.float32)
    for i in range(16):
      for jj in range(H // 16):
        zb[i, pl.ds(jj * 16, 16)] = zero16
    for k in range(rz // 16):
      pltpu.sync_copy(zb, acc.at[pl.ds(w * rz + k * 16, 16)])
    plsc.subcore_barrier()

    def block(jb, carry):
      pltpu.sync_copy(gidx.at[w, jb], gbB)
      pltpu.sync_copy(sidx.at[w, jb], sbB)
      for gj in range(BLK // KB):
        gds = []
        for b in range(KB):
          gds.append(pltpu.async_copy(table.at[gbB.at[gj * KB + b]],
                                      rows[b], gsem[b]))
        sds = []
        for b in range(KB):
          gds[b].wait()
          sds.append(pltpu.async_copy(rows[b],
                                      acc.at[sbB.at[gj * KB + b]],
                                      ssem[b], add=True))
        for b in range(KB):
          sds[b].wait()
      return carry

    lax.fori_loop(0, nblk, block, 0)
    plsc.subcore_barrier()
    pltpu.sync_copy(acc.at[pl.ds(s * rwb, rwb)],
                    out.at[c, pl.ds(s * rwb, rwb)])

  return seg


_seg_gcn = _make_segsum(NCH_E, N_ACC, N_WB)
_seg_t = _make_segsum(NCH_ES, NS_WB, NS_WB)
_seg_f = _make_segsum(NCH_ES, N_ACC, N_WB)


def _make_counts():
  mesh = plsc.VectorSubcoreMesh(core_axis_name="c", subcore_axis_name="s")

  @functools.partial(
      pl.kernel,
      out_type=[
          jax.ShapeDtypeStruct((NC, N_WB, 16), jnp.float32),
          jax.ShapeDtypeStruct((NC, NS_WB, 16), jnp.float32),
          jax.ShapeDtypeStruct((NC, N_WB, 16), jnp.float32),
      ],
      mesh=mesh,
      scratch_types=[
          pltpu.VMEM((NCH_E, CH), jnp.int32),
          pltpu.VMEM((NCH_ES, CH), jnp.int32),
          pltpu.VMEM((NCH_ES, CH), jnp.int32),
          pltpu.VMEM((CH, 16), jnp.float32),
          pltpu.VMEM((16, 16), jnp.float32),
          pltpu.VMEM_SHARED((N_ACC, 16), jnp.float32),
          pltpu.VMEM_SHARED((NS_WB, 16), jnp.float32),
          pltpu.VMEM_SHARED((N_ACC, 16), jnp.float32),
          pltpu.SemaphoreType.DMA,
      ],
  )
  def cnt(dste, fdst, fsrc, o_deg, o_cf, o_cn, vd, vf, vn, ones, zb,
          a_deg, a_cf, a_cn, sem):
    c = lax.axis_index("c")
    s = lax.axis_index("s")
    w = c * NSUB + s
    one16 = jnp.ones((16,), jnp.float32)
    zero16 = jnp.zeros((16,), jnp.float32)
    for i in range(CH):
      ones[i, pl.ds(0, 16)] = one16
    for i in range(16):
      zb[i, pl.ds(0, 16)] = zero16
    rzn = N_ACC // NW
    rzf = NS_WB // NW
    for tgt, rzz in ((a_deg, rzn), (a_cn, rzn), (a_cf, rzf)):
      for k in range(rzz // 16):
        pltpu.sync_copy(zb, tgt.at[pl.ds(w * rzz + k * 16, 16)])
    pltpu.sync_copy(dste.at[w], vd)
    pltpu.sync_copy(fdst.at[w], vf)
    pltpu.sync_copy(fsrc.at[w], vn)
    plsc.subcore_barrier()

    def mk_body(vref, acc_ref):
      def body(ch, carry):
        pltpu.sync_copy(ones, acc_ref.at[vref.at[ch]], add=True)
        return carry
      return body

    lax.fori_loop(0, NCH_E, mk_body(vd, a_deg), 0)
    lax.fori_loop(0, NCH_ES, mk_body(vf, a_cf), 0)
    lax.fori_loop(0, NCH_ES, mk_body(vn, a_cn), 0)
    plsc.subcore_barrier()
    rn = N_WB // NSUB
    rf = NS_WB // NSUB
    pltpu.sync_copy(a_deg.at[pl.ds(s * rn, rn)],
                    o_deg.at[c, pl.ds(s * rn, rn)])
    pltpu.sync_copy(a_cf.at[pl.ds(s * rf, rf)],
                    o_cf.at[c, pl.ds(s * rf, rf)])
    pltpu.sync_copy(a_cn.at[pl.ds(s * rn, rn)],
                    o_cn.at[c, pl.ds(s * rn, rn)])

  return cnt


_counts = _make_counts()

BR = 1000


def _mm_bias(x, w, b):
  r, k = x.shape
  m = w.shape[1]

  def body(x_ref, w_ref, b_ref, o_ref):
    o_ref[...] = jnp.dot(x_ref[...], w_ref[...],
                         preferred_element_type=jnp.float32) + b_ref[...]

  return pl.pallas_call(
      body,
      grid=(r // BR,),
      in_specs=[
          pl.BlockSpec((BR, k), lambda i: (i, 0)),
          pl.BlockSpec((k, m), lambda i: (0, 0)),
          pl.BlockSpec((1, m), lambda i: (0, 0)),
      ],
      out_specs=pl.BlockSpec((BR, m), lambda i: (i, 0)),
      out_shape=jax.ShapeDtypeStruct((r, m), jnp.float32),
  )(x, w, b.reshape(1, m))


def _pre_gcn(h, w, dinv):
  def body(h_ref, w_ref, d_ref, o_ref):
    o_ref[...] = jnp.dot(h_ref[...], w_ref[...],
                         preferred_element_type=jnp.float32) * d_ref[...]

  return pl.pallas_call(
      body,
      grid=(N // BR,),
      in_specs=[
          pl.BlockSpec((BR, H), lambda i: (i, 0)),
          pl.BlockSpec((H, H), lambda i: (0, 0)),
          pl.BlockSpec((BR, 1), lambda i: (i, 0)),
      ],
      out_specs=pl.BlockSpec((BR, H), lambda i: (i, 0)),
      out_shape=jax.ShapeDtypeStruct((N, H), jnp.float32),
  )(h, w, dinv)


def _post_gcn(p0, p1, scaled, dinv, a, c):
  def body(p0_ref, p1_ref, s_ref, d_ref, a_ref, c_ref, o_ref):
    agg = (p0_ref[...] + p1_ref[...] + s_ref[...]) * d_ref[...]
    o_ref[...] = jnp.maximum(agg * a_ref[...] + c_ref[...], 0.0)

  return pl.pallas_call(
      body,
      grid=(N // BR,),
      in_specs=[
          pl.BlockSpec((BR, H), lambda i: (i, 0)),
          pl.BlockSpec((BR, H), lambda i: (i, 0)),
          pl.BlockSpec((BR, H), lambda i: (i, 0)),
          pl.BlockSpec((BR, 1), lambda i: (i, 0)),
          pl.BlockSpec((1, H), lambda i: (0, 0)),
          pl.BlockSpec((1, H), lambda i: (0, 0)),
      ],
      out_specs=pl.BlockSpec((BR, H), lambda i: (i, 0)),
      out_shape=jax.ShapeDtypeStruct((N, H), jnp.float32),
  )(p0, p1, scaled, dinv, a.reshape(1, H), c.reshape(1, H))


def _sage_t(p0, p1, rcf, wl, hs, wr, a, c):
  def body(p0_ref, p1_ref, r_ref, wl_ref, hs_ref, wr_ref, a_ref, c_ref,
           o_ref):
    mean = (p0_ref[...] + p1_ref[...]) * r_ref[...]
    hsv = hs_ref[:, :HS]
    t = (jnp.dot(mean, wl_ref[...], preferred_element_type=jnp.float32)
         + jnp.dot(hsv, wr_ref[...], preferred_element_type=jnp.float32))
    res = jnp.maximum(t * a_ref[...] + c_ref[...], 0.0) + hsv
    o_ref[...] = jnp.concatenate([res, jnp.zeros_like(res)], axis=1)

  return pl.pallas_call(
      body,
      grid=(NS // BR,),
      in_specs=[
          pl.BlockSpec((BR, H), lambda i: (i, 0)),
          pl.BlockSpec((BR, H), lambda i: (i, 0)),
          pl.BlockSpec((BR, 1), lambda i: (i, 0)),
          pl.BlockSpec((H, HS), lambda i: (0, 0)),
          pl.BlockSpec((BR, H), lambda i: (i, 0)),
          pl.BlockSpec((HS, HS), lambda i: (0, 0)),
          pl.BlockSpec((1, HS), lambda i: (0, 0)),
          pl.BlockSpec((1, HS), lambda i: (0, 0)),
      ],
      out_specs=pl.BlockSpec((BR, H), lambda i: (i, 0)),
      out_shape=jax.ShapeDtypeStruct((NS, H), jnp.float32),
  )(p0, p1, rcf, wl, hs, wr, a.reshape(1, HS), c.reshape(1, HS))


def _sage_f(p0, p1, rcn, wl, hn, wr, bl, hc):
  def body(p0_ref, p1_ref, r_ref, wl_ref, hn_ref, wr_ref, b_ref, hc_ref,
           o_ref):
    mean = (p0_ref[:, :HS] + p1_ref[:, :HS]) * r_ref[...]
    o_ref[...] = (jnp.dot(mean, wl_ref[...],
                          preferred_element_type=jnp.float32)
                  + jnp.dot(hn_ref[...], wr_ref[...],
                            preferred_element_type=jnp.float32)
                  + b_ref[...] + hc_ref[...])

  return pl.pallas_call(
      body,
      grid=(N // BR,),
      in_specs=[
          pl.BlockSpec((BR, H), lambda i: (i, 0)),
          pl.BlockSpec((BR, H), lambda i: (i, 0)),
          pl.BlockSpec((BR, 1), lambda i: (i, 0)),
          pl.BlockSpec((HS, H), lambda i: (0, 0)),
          pl.BlockSpec((BR, H), lambda i: (i, 0)),
          pl.BlockSpec((H, H), lambda i: (0, 0)),
          pl.BlockSpec((1, H), lambda i: (0, 0)),
          pl.BlockSpec((BR, H), lambda i: (i, 0)),
      ],
      out_specs=pl.BlockSpec((BR, H), lambda i: (i, 0)),
      out_shape=jax.ShapeDtypeStruct((N, H), jnp.float32),
  )(p0, p1, rcn, wl, hn, wr, bl.reshape(1, H), hc)


def _readout(h, w0, b0, w1, b1, w2p, b2p):
  def body(h_ref, w0_ref, b0_ref, w1_ref, b1_ref, w2_ref, b2_ref, o_ref):
    t = jnp.maximum(jnp.dot(h_ref[...], w0_ref[...],
                            preferred_element_type=jnp.float32)
                    + b0_ref[...], 0.0)
    t = jnp.maximum(jnp.dot(t, w1_ref[...],
                            preferred_element_type=jnp.float32)
                    + b1_ref[...], 0.0)
    o_ref[...] = jnp.dot(t, w2_ref[...],
                         preferred_element_type=jnp.float32) + b2_ref[...]

  return pl.pallas_call(
      body,
      grid=(N // BR,),
      in_specs=[
          pl.BlockSpec((BR, H), lambda i: (i, 0)),
          pl.BlockSpec((H, H // 2), lambda i: (0, 0)),
          pl.BlockSpec((1, H // 2), lambda i: (0, 0)),
          pl.BlockSpec((H // 2, H // 4), lambda i: (0, 0)),
          pl.BlockSpec((1, H // 4), lambda i: (0, 0)),
          pl.BlockSpec((H // 4, H), lambda i: (0, 0)),
          pl.BlockSpec((1, H), lambda i: (0, 0)),
      ],
      out_specs=pl.BlockSpec((BR, H), lambda i: (i, 0)),
      out_shape=jax.ShapeDtypeStruct((N, H), jnp.float32),
  )(h, w0, b0.reshape(1, H // 2), w1, b1.reshape(1, H // 4), w2p,
    b2p.reshape(1, H))


def _dinv_from_counts(c0, c1):
  r = c0.shape[0]

  def body(a_ref, b_ref, o_ref):
    o_ref[...] = lax.rsqrt(a_ref[:, 0:1] + b_ref[:, 0:1] + 1.0)

  return pl.pallas_call(
      body,
      grid=(r // BR,),
      in_specs=[
          pl.BlockSpec((BR, 16), lambda i: (i, 0)),
          pl.BlockSpec((BR, 16), lambda i: (i, 0)),
      ],
      out_specs=pl.BlockSpec((BR, 1), lambda i: (i, 0)),
      out_shape=jax.ShapeDtypeStruct((r, 1), jnp.float32),
  )(c0, c1)


def _rcnt_from_counts(c0, c1):
  r = c0.shape[0]

  def body(a_ref, b_ref, o_ref):
    o_ref[...] = 1.0 / jnp.maximum(a_ref[:, 0:1] + b_ref[:, 0:1], 1.0)

  return pl.pallas_call(
      body,
      grid=(r // BR,),
      in_specs=[
          pl.BlockSpec((BR, 16), lambda i: (i, 0)),
          pl.BlockSpec((BR, 16), lambda i: (i, 0)),
      ],
      out_specs=pl.BlockSpec((BR, 1), lambda i: (i, 0)),
      out_shape=jax.ShapeDtypeStruct((r, 1), jnp.float32),
  )(c0, c1)


def _pad3d(idx, total, fill):
  return jnp.concatenate(
      [idx, jnp.full((total - idx.shape[0],), fill, jnp.int32)]
  ).reshape(NW, total // (NW * CH), CH)


def _pad4d(idx, total, fill):
  return jnp.concatenate(
      [idx, jnp.full((total - idx.shape[0],), fill, jnp.int32)]
  ).reshape(NW, total // (NW * BLK * CH), BLK, CH)


def kernel(x, x_batch, fragments, edge_index, fragments_edge_index, enc_W,
           enc_b, encs_W, encs_b, gcn_W, gcn_b, ts_Wl, ts_bl, ts_Wr, fs_Wl,
           fs_bl, fs_Wr, bn_g, bn_b, bns_g, bns_b, ro_W0, ro_b0, ro_W1,
           ro_b1, ro_W2, ro_b2):
  src = edge_index[0]
  dst = edge_index[1]
  fsrc = fragments_edge_index[0]
  fdst = fragments_edge_index[1]

  srcE_g = _pad4d(src, E_PAD, 0)
  dstE_s = _pad4d(dst, E_PAD, N)
  fsrc_g = _pad4d(fsrc, ES_PAD, 0)
  fsrc_s = _pad4d(fsrc, ES_PAD, N)
  fdst_g = _pad4d(fdst, ES_PAD, 0)
  fdst_s = _pad4d(fdst, ES_PAD, NS)
  cN, cF, cN2 = _counts(_pad3d(dst, E_PAD, N), _pad3d(fdst, ES_PAD, NS),
                        _pad3d(fsrc, ES_PAD, N))
  dinv = _dinv_from_counts(cN[0, :N], cN[1, :N])
  rcf = _rcnt_from_counts(cF[0, :NS], cF[1, :NS])
  rcn = _rcnt_from_counts(cN2[0, :N], cN2[1, :N])

  bn_scale = 1.0 / jnp.sqrt(1.0 + 1e-05)
  h = _mm_bias(x, enc_W, enc_b)
  encs_Wp = jnp.zeros((encs_W.shape[0], H), jnp.float32).at[:, :HS].set(
      encs_W)
  encs_bp = jnp.zeros((H,), jnp.float32).at[:HS].set(encs_b)
  hs = _mm_bias(fragments, encs_Wp, encs_bp)

  for i in range(L):
    h_c = h
    a_g = bn_g[i] * bn_scale
    c_g = gcn_b[i] * a_g + bn_b[i]
    a_s = bns_g[i] * bn_scale
    c_s = ts_bl[i] * a_s + bns_b[i]

    scaled = _pre_gcn(h, gcn_W[i], dinv)
    p = _seg_gcn(scaled, srcE_g, dstE_s)
    h_new = _post_gcn(p[0, :N], p[1, :N], scaled, dinv, a_g, c_g)

    t = _seg_t(h_new, fsrc_g, fdst_s)
    hs = _sage_t(t[0, :NS], t[1, :NS], rcf, ts_Wl[i], hs, ts_Wr[i], a_s,
                 c_s)

    f = _seg_f(hs, fdst_g, fsrc_s)
    h = _sage_f(f[0, :N], f[1, :N], rcn, fs_Wl[i], h_new, fs_Wr[i],
                fs_bl[i], h_c)

  w2p = jnp.zeros((H // 4, H), jnp.float32).at[:, :10].set(ro_W2)
  b2p = jnp.zeros((H,), jnp.float32).at[:10].set(ro_b2)
  y = _readout(h, ro_W0, ro_b0, ro_W1, ro_b1, w2p, b2p)
  return y[:, :10]


# speedup vs baseline: 2.2696x; 2.2696x over previous
import functools

import jax
import jax.numpy as jnp
from jax import lax
from jax.experimental import pallas as pl
from jax.experimental.pallas import tpu as pltpu
from jax.experimental.pallas import tpu_sc as plsc

N = 10000
NS = 5000
H = 128
HS = 64
E = 320000
ES = 50000
L = 3

NC = 2
NSUB = 16
NW = NC * NSUB
CH = 128

NCH_E = 79
NCH_ES = 13
E_PAD = NW * NCH_E * CH
ES_PAD = NW * NCH_ES * CH

N_WB = 10112
NS_WB = 5120
N_ACC = 10240


def _make_segsum(nch, rpad, n_wb):
  mesh = plsc.VectorSubcoreMesh(core_axis_name="c", subcore_axis_name="s")
  rz = rpad // NW
  rwb = n_wb // NSUB

  @functools.partial(
      pl.kernel,
      out_type=jax.ShapeDtypeStruct((NC, n_wb, H), jnp.float32),
      mesh=mesh,
      scratch_types=[
          pltpu.VMEM((nch, CH), jnp.int32),
          pltpu.VMEM((nch, CH), jnp.int32),
          pltpu.VMEM((CH, H), jnp.float32),
          pltpu.VMEM((16, H), jnp.float32),
          pltpu.VMEM_SHARED((rpad, H), jnp.float32),
          pltpu.SemaphoreType.DMA,
      ],
  )
  def seg(table, gidx, sidx, out, vg, vs, rows, zb, acc, sem):
    c = lax.axis_index("c")
    s = lax.axis_index("s")
    w = c * NSUB + s
    zero16 = jnp.zeros((16,), jnp.float32)
    for i in range(16):
      for jj in range(H // 16):
        zb[i, pl.ds(jj * 16, 16)] = zero16
    for k in range(rz // 16):
      pltpu.sync_copy(zb, acc.at[pl.ds(w * rz + k * 16, 16)])
    pltpu.sync_copy(gidx.at[w], vg)
    pltpu.sync_copy(sidx.at[w], vs)
    plsc.subcore_barrier()

    def body(ch, carry):
      pltpu.async_copy(table.at[vg.at[ch]], rows, sem).wait()
      pltpu.sync_copy(rows, acc.at[vs.at[ch]], add=True)
      return carry

    lax.fori_loop(0, nch, body, 0)
    plsc.subcore_barrier()
    pltpu.sync_copy(acc.at[pl.ds(s * rwb, rwb)],
                    out.at[c, pl.ds(s * rwb, rwb)])

  return seg


_seg_gcn = _make_segsum(NCH_E, N_ACC, N_WB)
_seg_t = _make_segsum(NCH_ES, NS_WB, NS_WB)
_seg_f = _make_segsum(NCH_ES, N_ACC, N_WB)


def _make_counts():
  mesh = plsc.VectorSubcoreMesh(core_axis_name="c", subcore_axis_name="s")

  @functools.partial(
      pl.kernel,
      out_type=[
          jax.ShapeDtypeStruct((NC, N_WB, 16), jnp.float32),
          jax.ShapeDtypeStruct((NC, NS_WB, 16), jnp.float32),
          jax.ShapeDtypeStruct((NC, N_WB, 16), jnp.float32),
      ],
      mesh=mesh,
      scratch_types=[
          pltpu.VMEM((NCH_E, CH), jnp.int32),
          pltpu.VMEM((NCH_ES, CH), jnp.int32),
          pltpu.VMEM((NCH_ES, CH), jnp.int32),
          pltpu.VMEM((CH, 16), jnp.float32),
          pltpu.VMEM((16, 16), jnp.float32),
          pltpu.VMEM_SHARED((N_ACC, 16), jnp.float32),
          pltpu.VMEM_SHARED((NS_WB, 16), jnp.float32),
          pltpu.VMEM_SHARED((N_ACC, 16), jnp.float32),
          pltpu.SemaphoreType.DMA,
      ],
  )
  def cnt(dste, fdst, fsrc, o_deg, o_cf, o_cn, vd, vf, vn, ones, zb,
          a_deg, a_cf, a_cn, sem):
    c = lax.axis_index("c")
    s = lax.axis_index("s")
    w = c * NSUB + s
    one16 = jnp.ones((16,), jnp.float32)
    zero16 = jnp.zeros((16,), jnp.float32)
    for i in range(CH):
      ones[i, pl.ds(0, 16)] = one16
    for i in range(16):
      zb[i, pl.ds(0, 16)] = zero16
    rzn = N_ACC // NW
    rzf = NS_WB // NW
    for tgt, rzz in ((a_deg, rzn), (a_cn, rzn), (a_cf, rzf)):
      for k in range(rzz // 16):
        pltpu.sync_copy(zb, tgt.at[pl.ds(w * rzz + k * 16, 16)])
    pltpu.sync_copy(dste.at[w], vd)
    pltpu.sync_copy(fdst.at[w], vf)
    pltpu.sync_copy(fsrc.at[w], vn)
    plsc.subcore_barrier()

    def mk_body(vref, acc_ref):
      def body(ch, carry):
        pltpu.sync_copy(ones, acc_ref.at[vref.at[ch]], add=True)
        return carry
      return body

    lax.fori_loop(0, NCH_E, mk_body(vd, a_deg), 0)
    lax.fori_loop(0, NCH_ES, mk_body(vf, a_cf), 0)
    lax.fori_loop(0, NCH_ES, mk_body(vn, a_cn), 0)
    plsc.subcore_barrier()
    rn = N_WB // NSUB
    rf = NS_WB // NSUB
    pltpu.sync_copy(a_deg.at[pl.ds(s * rn, rn)],
                    o_deg.at[c, pl.ds(s * rn, rn)])
    pltpu.sync_copy(a_cf.at[pl.ds(s * rf, rf)],
                    o_cf.at[c, pl.ds(s * rf, rf)])
    pltpu.sync_copy(a_cn.at[pl.ds(s * rn, rn)],
                    o_cn.at[c, pl.ds(s * rn, rn)])

  return cnt


_counts = _make_counts()

BR = 1000


def _mm_bias(x, w, b):
  r, k = x.shape
  m = w.shape[1]

  def body(x_ref, w_ref, b_ref, o_ref):
    o_ref[...] = jnp.dot(x_ref[...], w_ref[...],
                         preferred_element_type=jnp.float32) + b_ref[...]

  return pl.pallas_call(
      body,
      grid=(r // BR,),
      in_specs=[
          pl.BlockSpec((BR, k), lambda i: (i, 0)),
          pl.BlockSpec((k, m), lambda i: (0, 0)),
          pl.BlockSpec((1, m), lambda i: (0, 0)),
      ],
      out_specs=pl.BlockSpec((BR, m), lambda i: (i, 0)),
      out_shape=jax.ShapeDtypeStruct((r, m), jnp.float32),
  )(x, w, b.reshape(1, m))


def _pre_gcn(h, w, dinv):
  def body(h_ref, w_ref, d_ref, o_ref):
    o_ref[...] = jnp.dot(h_ref[...], w_ref[...],
                         preferred_element_type=jnp.float32) * d_ref[...]

  return pl.pallas_call(
      body,
      grid=(N // BR,),
      in_specs=[
          pl.BlockSpec((BR, H), lambda i: (i, 0)),
          pl.BlockSpec((H, H), lambda i: (0, 0)),
          pl.BlockSpec((BR, 1), lambda i: (i, 0)),
      ],
      out_specs=pl.BlockSpec((BR, H), lambda i: (i, 0)),
      out_shape=jax.ShapeDtypeStruct((N, H), jnp.float32),
  )(h, w, dinv)


def _post_gcn(p0, p1, scaled, dinv, a, c):
  def body(p0_ref, p1_ref, s_ref, d_ref, a_ref, c_ref, o_ref):
    agg = (p0_ref[...] + p1_ref[...] + s_ref[...]) * d_ref[...]
    o_ref[...] = jnp.maximum(agg * a_ref[...] + c_ref[...], 0.0)

  return pl.pallas_call(
      body,
      grid=(N // BR,),
      in_specs=[
          pl.BlockSpec((BR, H), lambda i: (i, 0)),
          pl.BlockSpec((BR, H), lambda i: (i, 0)),
          pl.BlockSpec((BR, H), lambda i: (i, 0)),
          pl.BlockSpec((BR, 1), lambda i: (i, 0)),
          pl.BlockSpec((1, H), lambda i: (0, 0)),
          pl.BlockSpec((1, H), lambda i: (0, 0)),
      ],
      out_specs=pl.BlockSpec((BR, H), lambda i: (i, 0)),
      out_shape=jax.ShapeDtypeStruct((N, H), jnp.float32),
  )(p0, p1, scaled, dinv, a.reshape(1, H), c.reshape(1, H))


def _sage_t(p0, p1, rcf, wl, hs, wr, a, c):
  def body(p0_ref, p1_ref, r_ref, wl_ref, hs_ref, wr_ref, a_ref, c_ref,
           o_ref):
    mean = (p0_ref[...] + p1_ref[...]) * r_ref[...]
    hsv = hs_ref[:, :HS]
    t = (jnp.dot(mean, wl_ref[...], preferred_element_type=jnp.float32)
         + jnp.dot(hsv, wr_ref[...], preferred_element_type=jnp.float32))
    res = jnp.maximum(t * a_ref[...] + c_ref[...], 0.0) + hsv
    o_ref[...] = jnp.concatenate([res, jnp.zeros_like(res)], axis=1)

  return pl.pallas_call(
      body,
      grid=(NS // BR,),
      in_specs=[
          pl.BlockSpec((BR, H), lambda i: (i, 0)),
          pl.BlockSpec((BR, H), lambda i: (i, 0)),
          pl.BlockSpec((BR, 1), lambda i: (i, 0)),
          pl.BlockSpec((H, HS), lambda i: (0, 0)),
          pl.BlockSpec((BR, H), lambda i: (i, 0)),
          pl.BlockSpec((HS, HS), lambda i: (0, 0)),
          pl.BlockSpec((1, HS), lambda i: (0, 0)),
          pl.BlockSpec((1, HS), lambda i: (0, 0)),
      ],
      out_specs=pl.BlockSpec((BR, H), lambda i: (i, 0)),
      out_shape=jax.ShapeDtypeStruct((NS, H), jnp.float32),
  )(p0, p1, rcf, wl, hs, wr, a.reshape(1, HS), c.reshape(1, HS))


def _sage_f(p0, p1, rcn, wl, hn, wr, bl, hc):
  def body(p0_ref, p1_ref, r_ref, wl_ref, hn_ref, wr_ref, b_ref, hc_ref,
           o_ref):
    mean = (p0_ref[:, :HS] + p1_ref[:, :HS]) * r_ref[...]
    o_ref[...] = (jnp.dot(mean, wl_ref[...],
                          preferred_element_type=jnp.float32)
                  + jnp.dot(hn_ref[...], wr_ref[...],
                            preferred_element_type=jnp.float32)
                  + b_ref[...] + hc_ref[...])

  return pl.pallas_call(
      body,
      grid=(N // BR,),
      in_specs=[
          pl.BlockSpec((BR, H), lambda i: (i, 0)),
          pl.BlockSpec((BR, H), lambda i: (i, 0)),
          pl.BlockSpec((BR, 1), lambda i: (i, 0)),
          pl.BlockSpec((HS, H), lambda i: (0, 0)),
          pl.BlockSpec((BR, H), lambda i: (i, 0)),
          pl.BlockSpec((H, H), lambda i: (0, 0)),
          pl.BlockSpec((1, H), lambda i: (0, 0)),
          pl.BlockSpec((BR, H), lambda i: (i, 0)),
      ],
      out_specs=pl.BlockSpec((BR, H), lambda i: (i, 0)),
      out_shape=jax.ShapeDtypeStruct((N, H), jnp.float32),
  )(p0, p1, rcn, wl, hn, wr, bl.reshape(1, H), hc)


def _readout(h, w0, b0, w1, b1, w2p, b2p):
  def body(h_ref, w0_ref, b0_ref, w1_ref, b1_ref, w2_ref, b2_ref, o_ref):
    t = jnp.maximum(jnp.dot(h_ref[...], w0_ref[...],
                            preferred_element_type=jnp.float32)
                    + b0_ref[...], 0.0)
    t = jnp.maximum(jnp.dot(t, w1_ref[...],
                            preferred_element_type=jnp.float32)
                    + b1_ref[...], 0.0)
    o_ref[...] = jnp.dot(t, w2_ref[...],
                         preferred_element_type=jnp.float32) + b2_ref[...]

  return pl.pallas_call(
      body,
      grid=(N // BR,),
      in_specs=[
          pl.BlockSpec((BR, H), lambda i: (i, 0)),
          pl.BlockSpec((H, H // 2), lambda i: (0, 0)),
          pl.BlockSpec((1, H // 2), lambda i: (0, 0)),
          pl.BlockSpec((H // 2, H // 4), lambda i: (0, 0)),
          pl.BlockSpec((1, H // 4), lambda i: (0, 0)),
          pl.BlockSpec((H // 4, H), lambda i: (0, 0)),
          pl.BlockSpec((1, H), lambda i: (0, 0)),
      ],
      out_specs=pl.BlockSpec((BR, H), lambda i: (i, 0)),
      out_shape=jax.ShapeDtypeStruct((N, H), jnp.float32),
  )(h, w0, b0.reshape(1, H // 2), w1, b1.reshape(1, H // 4), w2p,
    b2p.reshape(1, H))


def _dinv_from_counts(c0, c1):
  r = c0.shape[0]

  def body(a_ref, b_ref, o_ref):
    o_ref[...] = lax.rsqrt(a_ref[:, 0:1] + b_ref[:, 0:1] + 1.0)

  return pl.pallas_call(
      body,
      grid=(r // BR,),
      in_specs=[
          pl.BlockSpec((BR, 16), lambda i: (i, 0)),
          pl.BlockSpec((BR, 16), lambda i: (i, 0)),
      ],
      out_specs=pl.BlockSpec((BR, 1), lambda i: (i, 0)),
      out_shape=jax.ShapeDtypeStruct((r, 1), jnp.float32),
  )(c0, c1)


def _rcnt_from_counts(c0, c1):
  r = c0.shape[0]

  def body(a_ref, b_ref, o_ref):
    o_ref[...] = 1.0 / jnp.maximum(a_ref[:, 0:1] + b_ref[:, 0:1], 1.0)

  return pl.pallas_call(
      body,
      grid=(r // BR,),
      in_specs=[
          pl.BlockSpec((BR, 16), lambda i: (i, 0)),
          pl.BlockSpec((BR, 16), lambda i: (i, 0)),
      ],
      out_specs=pl.BlockSpec((BR, 1), lambda i: (i, 0)),
      out_shape=jax.ShapeDtypeStruct((r, 1), jnp.float32),
  )(c0, c1)


def _pad3d(idx, total, fill):
  return jnp.concatenate(
      [idx, jnp.full((total - idx.shape[0],), fill, jnp.int32)]
  ).reshape(NW, total // (NW * CH), CH)


def kernel(x, x_batch, fragments, edge_index, fragments_edge_index, enc_W,
           enc_b, encs_W, encs_b, gcn_W, gcn_b, ts_Wl, ts_bl, ts_Wr, fs_Wl,
           fs_bl, fs_Wr, bn_g, bn_b, bns_g, bns_b, ro_W0, ro_b0, ro_W1,
           ro_b1, ro_W2, ro_b2):
  src = edge_index[0]
  dst = edge_index[1]
  fsrc = fragments_edge_index[0]
  fdst = fragments_edge_index[1]

  srcE_g = _pad3d(src, E_PAD, 0)
  dstE_s = _pad3d(dst, E_PAD, N)
  fsrc_g = _pad3d(fsrc, ES_PAD, 0)
  fsrc_s = _pad3d(fsrc, ES_PAD, N)
  fdst_g = _pad3d(fdst, ES_PAD, 0)
  fdst_s = _pad3d(fdst, ES_PAD, NS)
  cN, cF, cN2 = _counts(dstE_s, fdst_s, fsrc_s)
  dinv = _dinv_from_counts(cN[0, :N], cN[1, :N])
  rcf = _rcnt_from_counts(cF[0, :NS], cF[1, :NS])
  rcn = _rcnt_from_counts(cN2[0, :N], cN2[1, :N])

  bn_scale = 1.0 / jnp.sqrt(1.0 + 1e-05)
  h = _mm_bias(x, enc_W, enc_b)
  encs_Wp = jnp.zeros((encs_W.shape[0], H), jnp.float32).at[:, :HS].set(
      encs_W)
  encs_bp = jnp.zeros((H,), jnp.float32).at[:HS].set(encs_b)
  hs = _mm_bias(fragments, encs_Wp, encs_bp)

  for i in range(L):
    h_c = h
    a_g = bn_g[i] * bn_scale
    c_g = gcn_b[i] * a_g + bn_b[i]
    a_s = bns_g[i] * bn_scale
    c_s = ts_bl[i] * a_s + bns_b[i]

    scaled = _pre_gcn(h, gcn_W[i], dinv)
    p = _seg_gcn(scaled, srcE_g, dstE_s)
    h_new = _post_gcn(p[0, :N], p[1, :N], scaled, dinv, a_g, c_g)

    t = _seg_t(h_new, fsrc_g, fdst_s)
    hs = _sage_t(t[0, :NS], t[1, :NS], rcf, ts_Wl[i], hs, ts_Wr[i], a_s,
                 c_s)

    f = _seg_f(hs, fdst_g, fsrc_s)
    h = _sage_f(f[0, :N], f[1, :N], rcn, fs_Wl[i], h_new, fs_Wr[i],
                fs_bl[i], h_c)

  w2p = jnp.zeros((H // 4, H), jnp.float32).at[:, :10].set(ro_W2)
  b2p = jnp.zeros((H,), jnp.float32).at[:10].set(ro_b2)
  y = _readout(h, ro_W0, ro_b0, ro_W1, ro_b1, w2p, b2p)
  return y[:, :10]
